# Initial kernel scaffold; baseline (speedup 1.0000x reference)
import jax
import jax.numpy as jnp
from jax.experimental import pallas as pl


def kernel(x, emb, Vw, Vb, Ww, Wb):
    raise NotImplementedError("write your pallas kernel here")


# speedup vs baseline: 1.7407x; 1.7407x over previous
import functools

import jax
import jax.numpy as jnp
from jax import lax
from jax.experimental import pallas as pl
from jax.experimental.pallas import tpu as pltpu
from jax.experimental.pallas import tpu_sc as plsc

BATCH = 16384
SEQ = 200
EMB_DIM = 64
HIDDEN = 256
OUT = 2

NUM_CORES = 2
NUM_SUBCORES = 16
NUM_WORKERS = NUM_CORES * NUM_SUBCORES
ROWS_PER_WORKER = BATCH // NUM_WORKERS
HALF_SEQ = SEQ // 2


def _pool_body(x_hbm, emb_hbm, out_hbm, idx_v, rows_v, out_v, sem):
  wid = lax.axis_index("s") * NUM_CORES + lax.axis_index("c")
  row_base = wid * ROWS_PER_WORKER

  def row_step(r, carry):
    row = row_base + r
    pltpu.sync_copy(x_hbm.at[pl.ds(row * 2, 2)], idx_v)
    c0 = pltpu.make_async_copy(
        emb_hbm.at[idx_v.at[0]], rows_v.at[pl.ds(0, HALF_SEQ)], sem)
    c1 = pltpu.make_async_copy(
        emb_hbm.at[idx_v.at[1]], rows_v.at[pl.ds(HALF_SEQ, HALF_SEQ)], sem)
    c0.start()
    c1.start()
    c0.wait()
    c1.wait()

    def acc_step(j, accs):
      a0, a1, a2, a3 = accs
      a0 = a0 + rows_v[j, pl.ds(0, 16)]
      a1 = a1 + rows_v[j, pl.ds(16, 16)]
      a2 = a2 + rows_v[j, pl.ds(32, 16)]
      a3 = a3 + rows_v[j, pl.ds(48, 16)]
      return (a0, a1, a2, a3)

    z = jnp.zeros((16,), jnp.float32)
    a0, a1, a2, a3 = lax.fori_loop(0, SEQ, acc_step, (z, z, z, z))
    scale = jnp.float32(1.0 / SEQ)
    out_v[r, pl.ds(0, 16)] = a0 * scale
    out_v[r, pl.ds(16, 16)] = a1 * scale
    out_v[r, pl.ds(32, 16)] = a2 * scale
    out_v[r, pl.ds(48, 16)] = a3 * scale
    return carry

  lax.fori_loop(0, ROWS_PER_WORKER, row_step, 0)
  pltpu.sync_copy(out_v, out_hbm.at[pl.ds(row_base, ROWS_PER_WORKER)])


def _pool(x2, emb):
  mesh = plsc.VectorSubcoreMesh(core_axis_name="c", subcore_axis_name="s")
  return pl.kernel(
      _pool_body,
      out_type=jax.ShapeDtypeStruct((BATCH, EMB_DIM), jnp.float32),
      mesh=mesh,
      compiler_params=pltpu.CompilerParams(use_tc_tiling_on_sc=False),
      scratch_types=[
          pltpu.VMEM((2, HALF_SEQ), jnp.int32),
          pltpu.VMEM((SEQ, EMB_DIM), jnp.float32),
          pltpu.VMEM((ROWS_PER_WORKER, EMB_DIM), jnp.float32),
          pltpu.SemaphoreType.DMA,
      ],
  )(x2, emb)


def _mlp_body(h_ref, vwt_ref, vb_ref, wwt_ref, wb_ref, o_ref):
  h = h_ref[...]
  z = jnp.tanh(
      jnp.dot(h, vwt_ref[...], preferred_element_type=jnp.float32)
      + vb_ref[...])
  logits = (
      jnp.dot(z, wwt_ref[...], preferred_element_type=jnp.float32)
      + wb_ref[...])
  m = jnp.max(logits, axis=1, keepdims=True)
  lse = jnp.log(jnp.sum(jnp.exp(logits - m), axis=1, keepdims=True)) + m
  o_ref[...] = logits - lse


def _mlp(pooled, VwT, Vb2, WwT, Wb2):
  bb = 2048
  grid = (BATCH // bb,)
  return pl.pallas_call(
      _mlp_body,
      grid=grid,
      in_specs=[
          pl.BlockSpec((bb, EMB_DIM), lambda i: (i, 0)),
          pl.BlockSpec((EMB_DIM, HIDDEN), lambda i: (0, 0)),
          pl.BlockSpec((1, HIDDEN), lambda i: (0, 0)),
          pl.BlockSpec((HIDDEN, OUT), lambda i: (0, 0)),
          pl.BlockSpec((1, OUT), lambda i: (0, 0)),
      ],
      out_specs=pl.BlockSpec((bb, OUT), lambda i: (i, 0)),
      out_shape=jax.ShapeDtypeStruct((BATCH, OUT), jnp.float32),
  )(pooled, VwT, Vb2, WwT, Wb2)


@jax.jit
def kernel(x, emb, Vw, Vb, Ww, Wb):
  x2 = x.astype(jnp.int32).reshape(BATCH * 2, HALF_SEQ)
  pooled = _pool(x2, emb)
  return _mlp(pooled, Vw.T, Vb.reshape(1, HIDDEN), Ww.T, Wb.reshape(1, OUT))


# speedup vs baseline: 2.9554x; 1.6978x over previous
import functools

import jax
import jax.numpy as jnp
from jax import lax
from jax.experimental import pallas as pl
from jax.experimental.pallas import tpu as pltpu
from jax.experimental.pallas import tpu_sc as plsc

BATCH = 16384
SEQ = 200
EMB_DIM = 64
HIDDEN = 256
OUT = 2

NUM_CORES = 2
NUM_SUBCORES = 16
NUM_WORKERS = NUM_CORES * NUM_SUBCORES
ROWS_PER_WORKER = BATCH // NUM_WORKERS
HALF_SEQ = SEQ // 2


GRP = 64
GRP2 = 2 * GRP
NGRP = ROWS_PER_WORKER // GRP


def _pool_body(x_hbm, emb_hbm, out_hbm, idx_v, rows_v, out_v,
               sem_idx, sem0, sem1, sem2, sem3):
  wid = lax.axis_index("s") * NUM_CORES + lax.axis_index("c")
  row_base = wid * ROWS_PER_WORKER
  sems = (sem0, sem1, sem2, sem3)
  scale = jnp.float32(1.0 / SEQ)

  def gather_start(s, chunk, buf):
    pltpu.make_async_copy(
        emb_hbm.at[idx_v.at[s, chunk]], rows_v.at[buf], sems[buf]).start()

  def gather_wait(buf):
    pltpu.make_async_copy(
        emb_hbm.at[idx_v.at[0, 0]], rows_v.at[buf], sems[buf]).wait()

  def accum_store(b0, b1, out_row):
    def step(j, accs):
      a0, a1, a2, a3 = accs
      a0 = a0 + rows_v[b0, j, pl.ds(0, 16)] + rows_v[b1, j, pl.ds(0, 16)]
      a1 = a1 + rows_v[b0, j, pl.ds(16, 16)] + rows_v[b1, j, pl.ds(16, 16)]
      a2 = a2 + rows_v[b0, j, pl.ds(32, 16)] + rows_v[b1, j, pl.ds(32, 16)]
      a3 = a3 + rows_v[b0, j, pl.ds(48, 16)] + rows_v[b1, j, pl.ds(48, 16)]
      return (a0, a1, a2, a3)

    z = jnp.zeros((16,), jnp.float32)
    a0, a1, a2, a3 = lax.fori_loop(0, HALF_SEQ, step, (z, z, z, z))
    out_v[out_row, pl.ds(0, 16)] = a0 * scale
    out_v[out_row, pl.ds(16, 16)] = a1 * scale
    out_v[out_row, pl.ds(32, 16)] = a2 * scale
    out_v[out_row, pl.ds(48, 16)] = a3 * scale

  pltpu.sync_copy(x_hbm.at[pl.ds(row_base * 2, GRP2)], idx_v.at[0])
  pltpu.make_async_copy(
      x_hbm.at[pl.ds(row_base * 2 + GRP2, GRP2)], idx_v.at[1], sem_idx).start()

  for g in range(NGRP):
    s = g % 2
    if g >= 1:
      pltpu.make_async_copy(
          x_hbm.at[pl.ds(row_base * 2, GRP2)], idx_v.at[s], sem_idx).wait()
      if g + 1 < NGRP:
        pltpu.make_async_copy(
            x_hbm.at[pl.ds((row_base + (g + 1) * GRP) * 2, GRP2)],
            idx_v.at[(g + 1) % 2], sem_idx).start()

    for pr in range(2):
      gather_start(s, 2 * pr, 2 * pr)
      gather_start(s, 2 * pr + 1, 2 * pr + 1)

    def pair_body(p, carry, s=s, g=g):
      for pr in range(2):
        r = 2 * p + pr
        b0, b1 = 2 * pr, 2 * pr + 1
        gather_wait(b0)
        gather_wait(b1)
        accum_store(b0, b1, g * GRP + r)
        gather_start(s, 2 * (r + 2), b0)
        gather_start(s, 2 * (r + 2) + 1, b1)
      return carry

    lax.fori_loop(0, GRP // 2 - 1, pair_body, 0)

    for pr in range(2):
      b0, b1 = 2 * pr, 2 * pr + 1
      gather_wait(b0)
      gather_wait(b1)
      accum_store(b0, b1, g * GRP + GRP - 2 + pr)

  pltpu.sync_copy(out_v, out_hbm.at[pl.ds(row_base, ROWS_PER_WORKER)])


def _pool(x2, emb):
  mesh = plsc.VectorSubcoreMesh(core_axis_name="c", subcore_axis_name="s")
  return pl.kernel(
      _pool_body,
      out_type=jax.ShapeDtypeStruct((BATCH, EMB_DIM), jnp.float32),
      mesh=mesh,
      compiler_params=pltpu.CompilerParams(use_tc_tiling_on_sc=False),
      scratch_types=[
          pltpu.VMEM((2, GRP2, HALF_SEQ), jnp.int32),
          pltpu.VMEM((4, HALF_SEQ, EMB_DIM), jnp.float32),
          pltpu.VMEM((ROWS_PER_WORKER, EMB_DIM), jnp.float32),
          pltpu.SemaphoreType.DMA,
          pltpu.SemaphoreType.DMA,
          pltpu.SemaphoreType.DMA,
          pltpu.SemaphoreType.DMA,
          pltpu.SemaphoreType.DMA,
      ],
  )(x2, emb)


def _mlp_body(h_ref, vwt_ref, vb_ref, wwt_ref, wb_ref, o_ref):
  h = h_ref[...]
  z = jnp.tanh(
      jnp.dot(h, vwt_ref[...], preferred_element_type=jnp.float32)
      + vb_ref[...])
  logits = (
      jnp.dot(z, wwt_ref[...], preferred_element_type=jnp.float32)
      + wb_ref[...])
  m = jnp.max(logits, axis=1, keepdims=True)
  lse = jnp.log(jnp.sum(jnp.exp(logits - m), axis=1, keepdims=True)) + m
  o_ref[...] = logits - lse


def _mlp(pooled, VwT, Vb2, WwT, Wb2):
  bb = 2048
  grid = (BATCH // bb,)
  return pl.pallas_call(
      _mlp_body,
      grid=grid,
      in_specs=[
          pl.BlockSpec((bb, EMB_DIM), lambda i: (i, 0)),
          pl.BlockSpec((EMB_DIM, HIDDEN), lambda i: (0, 0)),
          pl.BlockSpec((1, HIDDEN), lambda i: (0, 0)),
          pl.BlockSpec((HIDDEN, OUT), lambda i: (0, 0)),
          pl.BlockSpec((1, OUT), lambda i: (0, 0)),
      ],
      out_specs=pl.BlockSpec((bb, OUT), lambda i: (i, 0)),
      out_shape=jax.ShapeDtypeStruct((BATCH, OUT), jnp.float32),
  )(pooled, VwT, Vb2, WwT, Wb2)


@jax.jit
def kernel(x, emb, Vw, Vb, Ww, Wb):
  x2 = x.astype(jnp.int32).reshape(BATCH * 2, HALF_SEQ)
  pooled = _pool(x2, emb)
  return _mlp(pooled, Vw.T, Vb.reshape(1, HIDDEN), Ww.T, Wb.reshape(1, OUT))


# speedup vs baseline: 3.7718x; 1.2762x over previous
import functools

import jax
import jax.numpy as jnp
from jax import lax
from jax.experimental import pallas as pl
from jax.experimental.pallas import tpu as pltpu
from jax.experimental.pallas import tpu_sc as plsc
from jax.experimental import layout as jx_layout

BATCH = 16384
SEQ = 200
EMB_DIM = 64
HIDDEN = 256
OUT = 2

NUM_CORES = 2
NUM_SUBCORES = 16
NUM_WORKERS = NUM_CORES * NUM_SUBCORES
ROWS_PER_WORKER = BATCH // NUM_WORKERS
HALF_SEQ = SEQ // 2


GRP = 64
GRP2 = 2 * GRP
NGRP = ROWS_PER_WORKER // GRP


def _pool_body(x_hbm, emb_hbm, out_hbm, idx_v, rows_v, out_v,
               sem_idx, sem0, sem1, sem2, sem3):
  wid = lax.axis_index("s") * NUM_CORES + lax.axis_index("c")
  row_base = wid * ROWS_PER_WORKER
  sems = (sem0, sem1, sem2, sem3)
  scale = jnp.float32(1.0 / SEQ)

  def gather_start(s, chunk, buf):
    pltpu.make_async_copy(
        emb_hbm.at[idx_v.at[s, chunk]], rows_v.at[buf], sems[buf]).start()

  def gather_wait(buf):
    pltpu.make_async_copy(
        emb_hbm.at[idx_v.at[0, 0]], rows_v.at[buf], sems[buf]).wait()

  def accum_store(b0, b1, out_row):
    def step(j, accs):
      a0, a1, a2, a3 = accs
      a0 = a0 + rows_v[b0, j, pl.ds(0, 16)] + rows_v[b1, j, pl.ds(0, 16)]
      a1 = a1 + rows_v[b0, j, pl.ds(16, 16)] + rows_v[b1, j, pl.ds(16, 16)]
      a2 = a2 + rows_v[b0, j, pl.ds(32, 16)] + rows_v[b1, j, pl.ds(32, 16)]
      a3 = a3 + rows_v[b0, j, pl.ds(48, 16)] + rows_v[b1, j, pl.ds(48, 16)]
      return (a0, a1, a2, a3)

    z = jnp.zeros((16,), jnp.float32)
    a0, a1, a2, a3 = lax.fori_loop(0, HALF_SEQ, step, (z, z, z, z))
    out_v[out_row, pl.ds(0, 16)] = a0 * scale
    out_v[out_row, pl.ds(16, 16)] = a1 * scale
    out_v[out_row, pl.ds(32, 16)] = a2 * scale
    out_v[out_row, pl.ds(48, 16)] = a3 * scale

  pltpu.sync_copy(x_hbm.at[pl.ds(row_base * 2, GRP2)], idx_v.at[0])
  pltpu.make_async_copy(
      x_hbm.at[pl.ds(row_base * 2 + GRP2, GRP2)], idx_v.at[1], sem_idx).start()

  for g in range(NGRP):
    s = g % 2
    if g >= 1:
      pltpu.make_async_copy(
          x_hbm.at[pl.ds(row_base * 2, GRP2)], idx_v.at[s], sem_idx).wait()
      if g + 1 < NGRP:
        pltpu.make_async_copy(
            x_hbm.at[pl.ds((row_base + (g + 1) * GRP) * 2, GRP2)],
            idx_v.at[(g + 1) % 2], sem_idx).start()

    for pr in range(2):
      gather_start(s, 2 * pr, 2 * pr)
      gather_start(s, 2 * pr + 1, 2 * pr + 1)

    def pair_body(p, carry, s=s, g=g):
      for pr in range(2):
        r = 2 * p + pr
        b0, b1 = 2 * pr, 2 * pr + 1
        gather_wait(b0)
        gather_wait(b1)
        accum_store(b0, b1, g * GRP + r)
        gather_start(s, 2 * (r + 2), b0)
        gather_start(s, 2 * (r + 2) + 1, b1)
      return carry

    lax.fori_loop(0, GRP // 2 - 1, pair_body, 0)

    for pr in range(2):
      b0, b1 = 2 * pr, 2 * pr + 1
      gather_wait(b0)
      gather_wait(b1)
      accum_store(b0, b1, g * GRP + GRP - 2 + pr)

  pltpu.sync_copy(out_v, out_hbm.at[pl.ds(row_base, ROWS_PER_WORKER)])


def _pool(x2, emb):
  mesh = plsc.VectorSubcoreMesh(core_axis_name="c", subcore_axis_name="s")
  return pl.kernel(
      _pool_body,
      out_type=jax.ShapeDtypeStruct((BATCH, EMB_DIM), jnp.float32),
      mesh=mesh,
      compiler_params=pltpu.CompilerParams(use_tc_tiling_on_sc=False),
      scratch_types=[
          pltpu.VMEM((2, GRP2, HALF_SEQ), jnp.int32),
          pltpu.VMEM((4, HALF_SEQ, EMB_DIM), jnp.float32),
          pltpu.VMEM((ROWS_PER_WORKER, EMB_DIM), jnp.float32),
          pltpu.SemaphoreType.DMA,
          pltpu.SemaphoreType.DMA,
          pltpu.SemaphoreType.DMA,
          pltpu.SemaphoreType.DMA,
          pltpu.SemaphoreType.DMA,
      ],
  )(x2, emb)


def _mlp_body(h_ref, vwt_ref, vb_ref, wwt_ref, wb_ref, o_ref):
  h = h_ref[...]
  z = jnp.tanh(
      jnp.dot(h, vwt_ref[...], preferred_element_type=jnp.float32)
      + vb_ref[...])
  logits = (
      jnp.dot(z, wwt_ref[...], preferred_element_type=jnp.float32)
      + wb_ref[...])
  m = jnp.max(logits, axis=1, keepdims=True)
  lse = jnp.log(jnp.sum(jnp.exp(logits - m), axis=1, keepdims=True)) + m
  o_ref[...] = logits - lse


def _mlp(pooled, VwT, Vb2, WwT, Wb2):
  bb = 2048
  grid = (BATCH // bb,)
  return pl.pallas_call(
      _mlp_body,
      grid=grid,
      in_specs=[
          pl.BlockSpec((bb, EMB_DIM), lambda i: (i, 0)),
          pl.BlockSpec((EMB_DIM, HIDDEN), lambda i: (0, 0)),
          pl.BlockSpec((1, HIDDEN), lambda i: (0, 0)),
          pl.BlockSpec((HIDDEN, OUT), lambda i: (0, 0)),
          pl.BlockSpec((1, OUT), lambda i: (0, 0)),
      ],
      out_specs=pl.BlockSpec((bb, OUT), lambda i: (i, 0)),
      out_shape=jax.ShapeDtypeStruct((BATCH, OUT), jnp.float32),
  )(pooled, VwT, Vb2, WwT, Wb2)


@jax.jit
def kernel(x, emb, Vw, Vb, Ww, Wb):
  x2 = x.astype(jnp.int32).reshape(BATCH * 2, HALF_SEQ)
  emb_lin = jx_layout.with_layout_constraint(
      emb, jx_layout.Layout(major_to_minor=(0, 1), tiling=()))
  pooled = _pool(x2, emb_lin)
  return _mlp(pooled, Vw.T, Vb.reshape(1, HIDDEN), Ww.T, Wb.reshape(1, OUT))


# speedup vs baseline: 3.8023x; 1.0081x over previous
import functools

import jax
import jax.numpy as jnp
from jax import lax
from jax.experimental import pallas as pl
from jax.experimental.pallas import tpu as pltpu
from jax.experimental.pallas import tpu_sc as plsc
from jax.experimental import layout as jx_layout

BATCH = 16384
SEQ = 200
EMB_DIM = 64
HIDDEN = 256
OUT = 2

NUM_CORES = 2
NUM_SUBCORES = 16
NUM_WORKERS = NUM_CORES * NUM_SUBCORES
ROWS_PER_WORKER = BATCH // NUM_WORKERS
HALF_SEQ = SEQ // 2


GRP = 128
GRP2 = 2 * GRP
NGRP = ROWS_PER_WORKER // GRP


def _pool_body(x_hbm, emb_hbm, out_hbm, idx_v, rows_v, out_v,
               sem_idx, sem0, sem1, sem2, sem3):
  wid = lax.axis_index("s") * NUM_CORES + lax.axis_index("c")
  row_base = wid * ROWS_PER_WORKER
  sems = (sem0, sem1, sem2, sem3)
  scale = jnp.float32(1.0 / SEQ)

  def gather_start(s, chunk, buf):
    pltpu.make_async_copy(
        emb_hbm.at[idx_v.at[s, chunk]], rows_v.at[buf], sems[buf]).start()

  def gather_wait(buf):
    pltpu.make_async_copy(
        emb_hbm.at[idx_v.at[0, 0]], rows_v.at[buf], sems[buf]).wait()

  def accum_store(b0, b1, out_row):
    def step(j, accs):
      a0, a1, a2, a3 = accs
      a0 = a0 + rows_v[b0, j, pl.ds(0, 16)] + rows_v[b1, j, pl.ds(0, 16)]
      a1 = a1 + rows_v[b0, j, pl.ds(16, 16)] + rows_v[b1, j, pl.ds(16, 16)]
      a2 = a2 + rows_v[b0, j, pl.ds(32, 16)] + rows_v[b1, j, pl.ds(32, 16)]
      a3 = a3 + rows_v[b0, j, pl.ds(48, 16)] + rows_v[b1, j, pl.ds(48, 16)]
      return (a0, a1, a2, a3)

    z = jnp.zeros((16,), jnp.float32)
    a0, a1, a2, a3 = lax.fori_loop(0, HALF_SEQ, step, (z, z, z, z))
    out_v[out_row, pl.ds(0, 16)] = a0 * scale
    out_v[out_row, pl.ds(16, 16)] = a1 * scale
    out_v[out_row, pl.ds(32, 16)] = a2 * scale
    out_v[out_row, pl.ds(48, 16)] = a3 * scale

  pltpu.sync_copy(x_hbm.at[pl.ds(row_base * 2, GRP2)], idx_v.at[0])
  pltpu.make_async_copy(
      x_hbm.at[pl.ds(row_base * 2 + GRP2, GRP2)], idx_v.at[1], sem_idx).start()

  for g in range(NGRP):
    s = g % 2
    if g >= 1:
      pltpu.make_async_copy(
          x_hbm.at[pl.ds(row_base * 2, GRP2)], idx_v.at[s], sem_idx).wait()
      if g + 1 < NGRP:
        pltpu.make_async_copy(
            x_hbm.at[pl.ds((row_base + (g + 1) * GRP) * 2, GRP2)],
            idx_v.at[(g + 1) % 2], sem_idx).start()

    for pr in range(2):
      gather_start(s, 2 * pr, 2 * pr)
      gather_start(s, 2 * pr + 1, 2 * pr + 1)

    def pair_body(p, carry, s=s, g=g):
      for pr in range(2):
        r = 2 * p + pr
        b0, b1 = 2 * pr, 2 * pr + 1
        gather_wait(b0)
        gather_wait(b1)
        accum_store(b0, b1, g * GRP + r)
        gather_start(s, 2 * (r + 2), b0)
        gather_start(s, 2 * (r + 2) + 1, b1)
      return carry

    lax.fori_loop(0, GRP // 2 - 1, pair_body, 0)

    for pr in range(2):
      b0, b1 = 2 * pr, 2 * pr + 1
      gather_wait(b0)
      gather_wait(b1)
      accum_store(b0, b1, g * GRP + GRP - 2 + pr)

  pltpu.sync_copy(out_v, out_hbm.at[pl.ds(row_base, ROWS_PER_WORKER)])


def _pool(x2, emb):
  mesh = plsc.VectorSubcoreMesh(core_axis_name="c", subcore_axis_name="s")
  return pl.kernel(
      _pool_body,
      out_type=jax.ShapeDtypeStruct((BATCH, EMB_DIM), jnp.float32),
      mesh=mesh,
      compiler_params=pltpu.CompilerParams(use_tc_tiling_on_sc=False),
      scratch_types=[
          pltpu.VMEM((2, GRP2, HALF_SEQ), jnp.int32),
          pltpu.VMEM((4, HALF_SEQ, EMB_DIM), jnp.float32),
          pltpu.VMEM((ROWS_PER_WORKER, EMB_DIM), jnp.float32),
          pltpu.SemaphoreType.DMA,
          pltpu.SemaphoreType.DMA,
          pltpu.SemaphoreType.DMA,
          pltpu.SemaphoreType.DMA,
          pltpu.SemaphoreType.DMA,
      ],
  )(x2, emb)


def _mlp_body(h_ref, vwt_ref, vb_ref, wwt_ref, wb_ref, o_ref):
  h = h_ref[...]
  z = jnp.tanh(
      jnp.dot(h, vwt_ref[...], preferred_element_type=jnp.float32)
      + vb_ref[...])
  logits = (
      jnp.dot(z, wwt_ref[...], preferred_element_type=jnp.float32)
      + wb_ref[...])
  m = jnp.max(logits, axis=1, keepdims=True)
  lse = jnp.log(jnp.sum(jnp.exp(logits - m), axis=1, keepdims=True)) + m
  o_ref[...] = logits - lse


def _mlp(pooled, VwT, Vb2, WwT, Wb2):
  bb = 2048
  grid = (BATCH // bb,)
  return pl.pallas_call(
      _mlp_body,
      grid=grid,
      in_specs=[
          pl.BlockSpec((bb, EMB_DIM), lambda i: (i, 0)),
          pl.BlockSpec((EMB_DIM, HIDDEN), lambda i: (0, 0)),
          pl.BlockSpec((1, HIDDEN), lambda i: (0, 0)),
          pl.BlockSpec((HIDDEN, OUT), lambda i: (0, 0)),
          pl.BlockSpec((1, OUT), lambda i: (0, 0)),
      ],
      out_specs=pl.BlockSpec((bb, OUT), lambda i: (i, 0)),
      out_shape=jax.ShapeDtypeStruct((BATCH, OUT), jnp.float32),
  )(pooled, VwT, Vb2, WwT, Wb2)


@jax.jit
def kernel(x, emb, Vw, Vb, Ww, Wb):
  x2 = x.astype(jnp.int32).reshape(BATCH * 2, HALF_SEQ)
  emb_lin = jx_layout.with_layout_constraint(
      emb, jx_layout.Layout(major_to_minor=(0, 1), tiling=()))
  pooled = _pool(x2, emb_lin)
  return _mlp(pooled, Vw.T, Vb.reshape(1, HIDDEN), Ww.T, Wb.reshape(1, OUT))


# speedup vs baseline: 3.8079x; 1.0015x over previous
import jax
import jax.numpy as jnp
from jax import lax
from jax.experimental import pallas as pl
from jax.experimental.pallas import tpu as pltpu
from jax.experimental.pallas import tpu_sc as plsc
from jax.experimental import layout as jx_layout

BATCH = 16384
SEQ = 200
EMB_DIM = 64
HIDDEN = 256
OUT = 2

NUM_CORES = 2
NUM_SUBCORES = 16
NUM_WORKERS = NUM_CORES * NUM_SUBCORES
ROWS_PER_WORKER = BATCH // NUM_WORKERS
HALF_SEQ = SEQ // 2


GRP = 128
GRP2 = 2 * GRP
NGRP = ROWS_PER_WORKER // GRP


def _pool_body(x_hbm, emb_hbm, out_hbm, idx_v, rows_v, out_v,
               sem_idx, sem0, sem1, sem2, sem3):
  wid = lax.axis_index("s") * NUM_CORES + lax.axis_index("c")
  row_base = wid * ROWS_PER_WORKER
  sems = (sem0, sem1, sem2, sem3)
  scale = jnp.float32(1.0 / SEQ)

  def gather_start(s, chunk, buf):
    pltpu.make_async_copy(
        emb_hbm.at[idx_v.at[s, chunk]], rows_v.at[buf], sems[buf]).start()

  def gather_wait(buf):
    pltpu.make_async_copy(
        emb_hbm.at[idx_v.at[0, 0]], rows_v.at[buf], sems[buf]).wait()

  def accum_store(b0, b1, out_row):
    def step(j, accs):
      a0, a1, a2, a3 = accs
      a0 = a0 + rows_v[b0, j, pl.ds(0, 16)] + rows_v[b1, j, pl.ds(0, 16)]
      a1 = a1 + rows_v[b0, j, pl.ds(16, 16)] + rows_v[b1, j, pl.ds(16, 16)]
      a2 = a2 + rows_v[b0, j, pl.ds(32, 16)] + rows_v[b1, j, pl.ds(32, 16)]
      a3 = a3 + rows_v[b0, j, pl.ds(48, 16)] + rows_v[b1, j, pl.ds(48, 16)]
      return (a0, a1, a2, a3)

    z = jnp.zeros((16,), jnp.float32)
    a0, a1, a2, a3 = lax.fori_loop(0, HALF_SEQ, step, (z, z, z, z))
    out_v[out_row, pl.ds(0, 16)] = a0 * scale
    out_v[out_row, pl.ds(16, 16)] = a1 * scale
    out_v[out_row, pl.ds(32, 16)] = a2 * scale
    out_v[out_row, pl.ds(48, 16)] = a3 * scale

  pltpu.sync_copy(x_hbm.at[pl.ds(row_base * 2, GRP2)], idx_v.at[0])
  pltpu.make_async_copy(
      x_hbm.at[pl.ds(row_base * 2 + GRP2, GRP2)], idx_v.at[1], sem_idx).start()

  for g in range(NGRP):
    s = g % 2
    if g >= 1:
      pltpu.make_async_copy(
          x_hbm.at[pl.ds(row_base * 2, GRP2)], idx_v.at[s], sem_idx).wait()
      if g + 1 < NGRP:
        pltpu.make_async_copy(
            x_hbm.at[pl.ds((row_base + (g + 1) * GRP) * 2, GRP2)],
            idx_v.at[(g + 1) % 2], sem_idx).start()

    for pr in range(2):
      gather_start(s, 2 * pr, 2 * pr)
      gather_start(s, 2 * pr + 1, 2 * pr + 1)

    def pair_body(p, carry, s=s, g=g):
      for pr in range(2):
        r = 2 * p + pr
        b0, b1 = 2 * pr, 2 * pr + 1
        gather_wait(b0)
        gather_wait(b1)
        accum_store(b0, b1, g * GRP + r)
        gather_start(s, 2 * (r + 2), b0)
        gather_start(s, 2 * (r + 2) + 1, b1)
      return carry

    lax.fori_loop(0, GRP // 2 - 1, pair_body, 0)

    for pr in range(2):
      b0, b1 = 2 * pr, 2 * pr + 1
      gather_wait(b0)
      gather_wait(b1)
      accum_store(b0, b1, g * GRP + GRP - 2 + pr)

  pltpu.sync_copy(out_v, out_hbm.at[pl.ds(row_base, ROWS_PER_WORKER)])


def _pool(x2, emb):
  mesh = plsc.VectorSubcoreMesh(core_axis_name="c", subcore_axis_name="s")
  return pl.kernel(
      _pool_body,
      out_type=jax.ShapeDtypeStruct((BATCH, EMB_DIM), jnp.float32),
      mesh=mesh,
      compiler_params=pltpu.CompilerParams(use_tc_tiling_on_sc=False),
      scratch_types=[
          pltpu.VMEM((2, GRP2, HALF_SEQ), jnp.int32),
          pltpu.VMEM((4, HALF_SEQ, EMB_DIM), jnp.float32),
          pltpu.VMEM((ROWS_PER_WORKER, EMB_DIM), jnp.float32),
          pltpu.SemaphoreType.DMA,
          pltpu.SemaphoreType.DMA,
          pltpu.SemaphoreType.DMA,
          pltpu.SemaphoreType.DMA,
          pltpu.SemaphoreType.DMA,
      ],
  )(x2, emb)


def _mlp_body(h_ref, vwt_ref, vb_ref, wwt_ref, wb_ref, o_ref):
  h = h_ref[...]
  z = jnp.tanh(
      jnp.dot(h, vwt_ref[...], preferred_element_type=jnp.float32)
      + vb_ref[...])
  logits = (
      jnp.dot(z, wwt_ref[...], preferred_element_type=jnp.float32)
      + wb_ref[...])
  m = jnp.max(logits, axis=1, keepdims=True)
  lse = jnp.log(jnp.sum(jnp.exp(logits - m), axis=1, keepdims=True)) + m
  o_ref[...] = logits - lse


def _mlp(pooled, VwT, Vb2, WwT, Wb2):
  bb = 2048
  grid = (BATCH // bb,)
  return pl.pallas_call(
      _mlp_body,
      grid=grid,
      in_specs=[
          pl.BlockSpec((bb, EMB_DIM), lambda i: (i, 0)),
          pl.BlockSpec((EMB_DIM, HIDDEN), lambda i: (0, 0)),
          pl.BlockSpec((1, HIDDEN), lambda i: (0, 0)),
          pl.BlockSpec((HIDDEN, OUT), lambda i: (0, 0)),
          pl.BlockSpec((1, OUT), lambda i: (0, 0)),
      ],
      out_specs=pl.BlockSpec((bb, OUT), lambda i: (i, 0)),
      out_shape=jax.ShapeDtypeStruct((BATCH, OUT), jnp.float32),
  )(pooled, VwT, Vb2, WwT, Wb2)


@jax.jit
def kernel(x, emb, Vw, Vb, Ww, Wb):
  x2 = x.astype(jnp.int32).reshape(BATCH * 2, HALF_SEQ)
  emb_lin = jx_layout.with_layout_constraint(
      emb, jx_layout.Layout(major_to_minor=(0, 1), tiling=()))
  pooled = _pool(x2, emb_lin)
  return _mlp(pooled, Vw.T, Vb.reshape(1, HIDDEN), Ww.T, Wb.reshape(1, OUT))
